# Initial kernel scaffold; baseline (speedup 1.0000x reference)
import jax
import jax.numpy as jnp
from jax.experimental import pallas as pl


def kernel(actionprobs, bscores, predactions, is_term):
    raise NotImplementedError("write your pallas kernel here")


# speedup vs baseline: 13.4240x; 13.4240x over previous
import functools

import jax
import jax.numpy as jnp
from jax import lax
from jax.experimental import pallas as pl
from jax.experimental.pallas import tpu as pltpu
from jax.experimental.pallas import tpu_sc as plsc

B, K, V = 32, 8, 32000
L = 16
NROWS = B * K
NW = 32
ROWS_PER_W = NROWS // NW
NV = V // L
NCAND = K * L


def _sc_stage(x):
  mesh = plsc.VectorSubcoreMesh(core_axis_name="c", subcore_axis_name="s")

  @functools.partial(
      pl.kernel,
      mesh=mesh,
      out_type=[
          jax.ShapeDtypeStruct((NROWS, NCAND), jnp.float32),
          jax.ShapeDtypeStruct((NROWS, NCAND), jnp.int32),
          jax.ShapeDtypeStruct((NROWS, 2 * L), jnp.float32),
      ],
      scratch_types=[
          pltpu.VMEM((V,), jnp.float32),
          pltpu.VMEM((NCAND,), jnp.float32),
          pltpu.VMEM((NCAND,), jnp.int32),
          pltpu.VMEM((2 * L,), jnp.float32),
      ],
  )
  def sc_k(x_hbm, vals_hbm, ids_hbm, stats_hbm, row_v, vals_v, ids_v, stats_v):
    wid = lax.axis_index("s") * 2 + lax.axis_index("c")
    lane = lax.iota(jnp.int32, 16)
    neg_inf = jnp.float32(-jnp.inf)

    def do_row(r, _):
      row = wid * ROWS_PER_W + r
      pltpu.sync_copy(x_hbm.at[row], row_v)

      def body1(i, carry):
        Rs, Is = carry
        v = row_v[pl.ds(i * L, L)]
        idx = lane + i * L
        nR = []
        nI = []
        for j in range(K):
          c = v > Rs[j]
          nR.append(jnp.where(c, v, Rs[j]))
          nI.append(jnp.where(c, idx, Is[j]))
          v, idx = jnp.where(c, Rs[j], v), jnp.where(c, Is[j], idx)
        return (tuple(nR), tuple(nI))

      R0 = tuple(jnp.full((L,), neg_inf, jnp.float32) for _ in range(K))
      I0 = tuple(jnp.zeros((L,), jnp.int32) for _ in range(K))
      Rs, Is = lax.fori_loop(0, NV, body1, (R0, I0))

      m16 = Rs[0]

      def body2(i, s):
        v = row_v[pl.ds(i * L, L)]
        return s + jnp.exp(v - m16)

      s16 = lax.fori_loop(0, NV, body2, jnp.zeros((L,), jnp.float32))

      for j in range(K):
        vals_v[pl.ds(j * L, L)] = Rs[j]
        ids_v[pl.ds(j * L, L)] = Is[j]
      stats_v[pl.ds(0, L)] = m16
      stats_v[pl.ds(L, L)] = s16
      pltpu.sync_copy(vals_v, vals_hbm.at[row])
      pltpu.sync_copy(ids_v, ids_hbm.at[row])
      pltpu.sync_copy(stats_v, stats_hbm.at[row])
      return 0

    lax.fori_loop(0, ROWS_PER_W, do_row, 0)

  return sc_k(x)


def _tc_body(x_ref, cv_ref, ci_ref, st_ref, bs_ref, pa_ref, it_ref,
             out_x_ref, out_pred_ref, out_sc_ref):
  x = x_ref[0]
  cand_vals = cv_ref[0]
  cand_ids = ci_ref[0]
  stats = st_ref[0]
  bscores = bs_ref[0]
  pred = pa_ref[0]
  is_term = it_ref[0]

  m16 = stats[:, 0:L]
  s16 = stats[:, L:2 * L]
  m_row = jnp.max(m16, axis=1, keepdims=True)
  s_row = jnp.sum(jnp.exp(m16 - m_row) * s16, axis=1, keepdims=True)
  lse = m_row + jnp.log(s_row)
  big = jnp.int32(1 << 30)
  neg_inf = jnp.float32(-jnp.inf)

  work = cand_vals
  tv, ti = [], []
  for _ in range(K):
    m = jnp.max(work, axis=1, keepdims=True)
    hit = work == m
    idsel = jnp.min(jnp.where(hit, cand_ids, big), axis=1, keepdims=True)
    tv.append(m)
    ti.append(idsel)
    work = jnp.where(hit & (cand_ids == idsel), neg_inf, work)
  top_vals = jnp.concatenate(tv, axis=1) - lse
  top_ids = jnp.concatenate(ti, axis=1)

  slot = lax.broadcasted_iota(jnp.int32, (K, K), 1)
  term_scores = jnp.where(slot == 0, 0.0, neg_inf)
  top_vals = jnp.where(is_term > 0, term_scores, top_vals)
  scores = top_vals + bscores

  fidx = (lax.broadcasted_iota(jnp.int32, (K, K), 0) * K
          + lax.broadcasted_iota(jnp.int32, (K, K), 1))
  beam_iota = lax.broadcasted_iota(jnp.int32, (K, 1), 0)
  sel_scores = []
  sids = []
  aids = []
  w = scores
  for _ in range(K):
    m = jnp.max(w)
    hit = w == m
    s = jnp.min(jnp.where(hit, fidx, big))
    sel_scores.append(m)
    sids.append(s // K)
    aids.append(jnp.sum(jnp.where(fidx == s, top_ids, 0)))
    w = jnp.where(fidx == s, neg_inf, w)

  ns = jnp.zeros((K, 1), jnp.float32)
  sid_col = jnp.zeros((K, 1), jnp.int32)
  aid_col = jnp.zeros((K, 1), jnp.int32)
  for k in range(K):
    pick = beam_iota == k
    ns = jnp.where(pick, sel_scores[k], ns)
    sid_col = jnp.where(pick, sids[k], sid_col)
    aid_col = jnp.where(pick, aids[k], aid_col)
  out_sc_ref[0] = ns

  pred_g = pred
  for j in range(K):
    pred_g = jnp.where(sid_col == j, pred[j:j + 1, :], pred_g)
  out_pred_ref[0, :, pl.ds(0, 16)] = pred_g
  out_pred_ref[0, :, pl.ds(16, 1)] = aid_col

  onehot = (sid_col == lax.broadcasted_iota(jnp.int32, (K, K), 1)
            ).astype(jnp.float32)
  xg = lax.dot_general(onehot, x, (((1,), (0,)), ((), ())),
                       preferred_element_type=jnp.float32)
  lse_g = lax.dot_general(onehot, lse, (((1,), (0,)), ((), ())),
                          preferred_element_type=jnp.float32)
  out_x_ref[0] = xg - lse_g


def _tc_stage(x, cand_vals, cand_ids, stats, bscores, predactions, is_term):
  grid = (B,)
  bmap = lambda b: (b, 0, 0)
  return pl.pallas_call(
      _tc_body,
      grid=grid,
      in_specs=[
          pl.BlockSpec((1, K, V), bmap),
          pl.BlockSpec((1, K, NCAND), bmap),
          pl.BlockSpec((1, K, NCAND), bmap),
          pl.BlockSpec((1, K, 2 * L), bmap),
          pl.BlockSpec((1, K, 1), bmap),
          pl.BlockSpec((1, K, 16), bmap),
          pl.BlockSpec((1, K, 1), bmap),
      ],
      out_specs=[
          pl.BlockSpec((1, K, V), bmap),
          pl.BlockSpec((1, K, 17), bmap),
          pl.BlockSpec((1, K, 1), bmap),
      ],
      out_shape=[
          jax.ShapeDtypeStruct((B, K, V), jnp.float32),
          jax.ShapeDtypeStruct((B, K, 17), jnp.int32),
          jax.ShapeDtypeStruct((B, K, 1), jnp.float32),
      ],
  )(x, cand_vals, cand_ids, stats, bscores, predactions, is_term)


def kernel(actionprobs, bscores, predactions, is_term):
  x2 = actionprobs.reshape(NROWS, V)
  cand_vals, cand_ids, stats = _sc_stage(x2)
  out_x, out_pred, out_sc = _tc_stage(
      actionprobs,
      cand_vals.reshape(B, K, NCAND),
      cand_ids.reshape(B, K, NCAND),
      stats.reshape(B, K, 2 * L),
      bscores.reshape(B, K, 1),
      predactions,
      is_term.astype(jnp.int32).reshape(B, K, 1),
  )
  return (out_x, out_pred, out_sc.reshape(B, K))


# speedup vs baseline: 28.3548x; 2.1122x over previous
import functools

import jax
import jax.numpy as jnp
from jax import lax
from jax.experimental import pallas as pl
from jax.experimental.pallas import tpu as pltpu
from jax.experimental.pallas import tpu_sc as plsc

B, K, V = 32, 8, 32000
L = 16
NROWS = B * K
NW = 32
ROWS_PER_W = NROWS // NW
NV = V // L
GROUP = 8
NG = NV // GROUP
NCAND = K * L


def _insert(Rs, Is, v, idx):
  nR, nI = [], []
  for j in range(K):
    c = v > Rs[j]
    nR.append(jnp.where(c, v, Rs[j]))
    nI.append(jnp.where(c, idx, Is[j]))
    v, idx = jnp.where(c, Rs[j], v), jnp.where(c, Is[j], idx)
  return tuple(nR), tuple(nI)


def _sc_stage(x):
  mesh = plsc.VectorSubcoreMesh(core_axis_name="c", subcore_axis_name="s")

  @functools.partial(
      pl.kernel,
      mesh=mesh,
      compiler_params=pltpu.CompilerParams(needs_layout_passes=False),
      out_type=[
          jax.ShapeDtypeStruct((NROWS, NCAND), jnp.float32),
          jax.ShapeDtypeStruct((NROWS, NCAND), jnp.int32),
          jax.ShapeDtypeStruct((NROWS, L), jnp.float32),
      ],
      scratch_types=[
          pltpu.VMEM((V,), jnp.float32),
          pltpu.VMEM((NCAND,), jnp.float32),
          pltpu.VMEM((NCAND,), jnp.int32),
          pltpu.VMEM((L,), jnp.float32),
      ],
  )
  def sc_k(x_hbm, vals_hbm, ids_hbm, s_hbm, row_v, vals_v, ids_v, s_v):
    wid = lax.axis_index("s") * 2 + lax.axis_index("c")
    lane = lax.iota(jnp.int32, 16)
    neg_inf = jnp.float32(-jnp.inf)

    def do_row(r, _):
      row = wid * ROWS_PER_W + r
      pltpu.sync_copy(x_hbm.at[row], row_v)

      def group_body(g, carry):
        Rs, Is, sa, sb, th = carry
        base = g * (GROUP * L)
        vs = [row_v[pl.ds(base + t * L, L)] for t in range(GROUP)]
        for t in range(0, GROUP, 2):
          sa = sa + jnp.exp(vs[t])
          sb = sb + jnp.exp(vs[t + 1])
        gmax = vs[0]
        for t in range(1, GROUP):
          gmax = jnp.maximum(gmax, vs[t])
        hit = jnp.any(gmax > th)

        def slow(args):
          Rs, Is, th = args
          for t in range(GROUP):
            chunk_hit = jnp.any(vs[t] > th)

            def ins(a):
              R2, I2 = _insert(a[0], a[1], vs[t],
                               lane + (g * GROUP + t) * L)
              return R2, I2

            Rs, Is = lax.cond(chunk_hit, ins, lambda a: a, (Rs, Is))
          sk = plsc.sort_key_val(Rs[0], lane, descending=True)[0]
          return Rs, Is, sk[K - 1]

        Rs, Is, th = lax.cond(hit, slow, lambda a: a, (Rs, Is, th))
        return Rs, Is, sa, sb, th

      R0 = tuple(jnp.full((L,), neg_inf, jnp.float32) for _ in range(K))
      I0 = tuple(jnp.zeros((L,), jnp.int32) for _ in range(K))
      z = jnp.zeros((L,), jnp.float32)
      Rs, Is, sa, sb, _ = lax.fori_loop(
          0, NG, group_body, (R0, I0, z, z, neg_inf))

      for j in range(K):
        vals_v[pl.ds(j * L, L)] = Rs[j]
        ids_v[pl.ds(j * L, L)] = Is[j]
      s_v[...] = sa + sb
      pltpu.sync_copy(vals_v, vals_hbm.at[row])
      pltpu.sync_copy(ids_v, ids_hbm.at[row])
      pltpu.sync_copy(s_v, s_hbm.at[row])
      return 0

    lax.fori_loop(0, ROWS_PER_W, do_row, 0)

  return sc_k(x)


def _sel_body(cv_ref, ci_ref, s_ref, bs_ref, pa_ref, it_ref,
              oh_ref, lse_ref, pred_ref, sc_ref):
  cand_vals = cv_ref[...]
  cand_ids = ci_ref[...]
  s16 = s_ref[...]
  bscores = bs_ref[...]
  pred = pa_ref[...]
  is_term = it_ref[...]

  big = jnp.int32(1 << 30)
  neg_inf = jnp.float32(-jnp.inf)

  lse = jnp.log(jnp.sum(s16, axis=2, keepdims=True))

  work = cand_vals
  tv, ti = [], []
  for _ in range(K):
    m = jnp.max(work, axis=2, keepdims=True)
    hitm = work == m
    idsel = jnp.min(jnp.where(hitm, cand_ids, big), axis=2, keepdims=True)
    tv.append(m)
    ti.append(idsel)
    work = jnp.where(hitm & (cand_ids == idsel), neg_inf, work)
  top_vals = jnp.concatenate(tv, axis=2) - lse
  top_ids = jnp.concatenate(ti, axis=2)

  slot = lax.broadcasted_iota(jnp.int32, (B, K, K), 2)
  term_scores = jnp.where(slot == 0, 0.0, neg_inf)
  top_vals = jnp.where(is_term > 0, term_scores, top_vals)
  scores = top_vals + bscores

  fidx = (lax.broadcasted_iota(jnp.int32, (B, K, K), 1) * K
          + slot)
  w = scores
  ns = jnp.zeros((B, K, 1), jnp.float32)
  sid = jnp.zeros((B, K, 1), jnp.int32)
  aid = jnp.zeros((B, K, 1), jnp.int32)
  kslot = lax.broadcasted_iota(jnp.int32, (B, K, 1), 1)
  for k in range(K):
    m2 = jnp.max(w, axis=2, keepdims=True)
    m = jnp.max(m2, axis=1, keepdims=True)
    hitm = w == m
    s2 = jnp.min(jnp.where(hitm, fidx, big), axis=2, keepdims=True)
    s = jnp.min(s2, axis=1, keepdims=True)
    a2 = jnp.sum(jnp.where(fidx == s, top_ids, 0), axis=2, keepdims=True)
    a = jnp.sum(a2, axis=1, keepdims=True)
    pick = kslot == k
    ns = jnp.where(pick, m, ns)
    sid = jnp.where(pick, s // K, sid)
    aid = jnp.where(pick, a, aid)
    w = jnp.where(fidx == s, neg_inf, w)
  sc_ref[...] = ns

  onehot = (sid == slot).astype(jnp.float32)
  oh_ref[...] = onehot

  lse_g = jnp.zeros((B, K, 1), jnp.float32)
  pred_g = pred
  for j in range(K):
    pick = sid == j
    lse_g = jnp.where(pick, lse[:, j:j + 1, :], lse_g)
    pred_g = jnp.where(pick, pred[:, j:j + 1, :], pred_g)
  lse_ref[...] = lse_g
  pred_ref[:, :, pl.ds(0, 16)] = pred_g
  pred_ref[:, :, pl.ds(16, 1)] = aid


def _sel_stage(cand_vals, cand_ids, s16, bscores, predactions, is_term):
  return pl.pallas_call(
      _sel_body,
      out_shape=[
          jax.ShapeDtypeStruct((B, K, K), jnp.float32),
          jax.ShapeDtypeStruct((B, K, 1), jnp.float32),
          jax.ShapeDtypeStruct((B, K, 17), jnp.int32),
          jax.ShapeDtypeStruct((B, K, 1), jnp.float32),
      ],
  )(cand_vals, cand_ids, s16, bscores, predactions, is_term)


def _gather_body(x_ref, oh_ref, lse_ref, out_ref):
  onehot = oh_ref[0]
  x = x_ref[0]
  lse_g = lse_ref[...]
  xg = lax.dot_general(onehot, x, (((1,), (0,)), ((), ())),
                       preferred_element_type=jnp.float32)
  out_ref[0] = xg - lse_g.reshape(K, 1)


def _gather_stage(x, onehot, lse_g):
  return pl.pallas_call(
      _gather_body,
      grid=(B,),
      in_specs=[
          pl.BlockSpec((1, K, V), lambda b: (b, 0, 0)),
          pl.BlockSpec((1, K, K), lambda b: (b, 0, 0)),
          pl.BlockSpec((1, K, 1), lambda b: (b, 0, 0)),
      ],
      out_specs=pl.BlockSpec((1, K, V), lambda b: (b, 0, 0)),
      out_shape=jax.ShapeDtypeStruct((B, K, V), jnp.float32),
  )(x, onehot, lse_g)


def kernel(actionprobs, bscores, predactions, is_term):
  x2 = actionprobs.reshape(NROWS, V)
  cand_vals, cand_ids, s16 = _sc_stage(x2)
  onehot, lse_g, new_pred, new_scores = _sel_stage(
      cand_vals.reshape(B, K, NCAND), cand_ids.reshape(B, K, NCAND),
      s16.reshape(B, K, L), bscores.reshape(B, K, 1), predactions,
      is_term.astype(jnp.int32).reshape(B, K, 1))
  out_x = _gather_stage(actionprobs, onehot, lse_g)
  return (out_x, new_pred, new_scores.reshape(B, K))


# speedup vs baseline: 29.7460x; 1.0491x over previous
import functools

import jax
import jax.numpy as jnp
from jax import lax
from jax.experimental import pallas as pl
from jax.experimental.pallas import tpu as pltpu
from jax.experimental.pallas import tpu_sc as plsc

B, K, V = 32, 8, 32000
L = 16
NROWS = B * K
NW = 32
ROWS_PER_W = NROWS // NW
NV = V // L
GROUP = 8
NG = NV // GROUP
NCAND = K * L


def _insert(Rs, Is, v, idx):
  nR, nI = [], []
  for j in range(K):
    c = v > Rs[j]
    nR.append(jnp.where(c, v, Rs[j]))
    nI.append(jnp.where(c, idx, Is[j]))
    v, idx = jnp.where(c, Rs[j], v), jnp.where(c, Is[j], idx)
  return tuple(nR), tuple(nI)


def _sc_stage(x):
  mesh = plsc.VectorSubcoreMesh(core_axis_name="c", subcore_axis_name="s")

  @functools.partial(
      pl.kernel,
      mesh=mesh,
      compiler_params=pltpu.CompilerParams(needs_layout_passes=False),
      out_type=[
          jax.ShapeDtypeStruct((NROWS * NCAND,), jnp.float32),
          jax.ShapeDtypeStruct((NROWS * NCAND,), jnp.int32),
          jax.ShapeDtypeStruct((NROWS * L,), jnp.float32),
      ],
      scratch_types=[
          pltpu.VMEM((V,), jnp.float32),
          pltpu.VMEM((V,), jnp.float32),
          pltpu.VMEM((ROWS_PER_W * NCAND,), jnp.float32),
          pltpu.VMEM((ROWS_PER_W * NCAND,), jnp.int32),
          pltpu.VMEM((ROWS_PER_W * L,), jnp.float32),
          pltpu.SemaphoreType.DMA,
          pltpu.SemaphoreType.DMA,
      ],
  )
  def sc_k(x_hbm, vals_hbm, ids_hbm, s_hbm, row_a, row_b, vals_v, ids_v,
           s_v, sem_a, sem_b):
    wid = lax.axis_index("s") * 2 + lax.axis_index("c")
    lane = lax.iota(jnp.int32, 16)
    neg_inf = jnp.float32(-jnp.inf)
    rowbase = wid * ROWS_PER_W

    bufs = [(row_a, sem_a), (row_b, sem_b)]
    handles = [pltpu.async_copy(x_hbm.at[rowbase], row_a, sem_a)]

    def make_row(row_v, r, handle, prefetch):
      handle.wait()
      if prefetch is not None:
        nbuf, nsem, nrow = prefetch
        handles.append(pltpu.async_copy(x_hbm.at[nrow], nbuf, nsem))

      def group_body(g, carry):
        Rs, Is, sa, sb, th = carry
        base = g * (GROUP * L)
        vs = [row_v[pl.ds(base + t * L, L)] for t in range(GROUP)]
        for t in range(0, GROUP, 2):
          sa = sa + jnp.exp(vs[t])
          sb = sb + jnp.exp(vs[t + 1])
        gmax = vs[0]
        for t in range(1, GROUP):
          gmax = jnp.maximum(gmax, vs[t])
        hit = jnp.any(gmax > th)

        def slow(args):
          Rs, Is, th = args
          for t in range(GROUP):
            chunk_hit = jnp.any(vs[t] > th)

            def ins(a):
              R2, I2 = _insert(a[0], a[1], vs[t],
                               lane + (g * GROUP + t) * L)
              return R2, I2

            Rs, Is = lax.cond(chunk_hit, ins, lambda a: a, (Rs, Is))
          sk = plsc.sort_key_val(Rs[0], lane, descending=True)[0]
          return Rs, Is, sk[K - 1]

        Rs, Is, th = lax.cond(hit, slow, lambda a: a, (Rs, Is, th))
        return Rs, Is, sa, sb, th

      R0 = tuple(jnp.full((L,), neg_inf, jnp.float32) for _ in range(K))
      I0 = tuple(jnp.zeros((L,), jnp.int32) for _ in range(K))
      z = jnp.zeros((L,), jnp.float32)
      Rs, Is, sa, sb, _ = lax.fori_loop(
          0, NG, group_body, (R0, I0, z, z, neg_inf))

      for j in range(K):
        vals_v[pl.ds(r * NCAND + j * L, L)] = Rs[j]
        ids_v[pl.ds(r * NCAND + j * L, L)] = Is[j]
      s_v[pl.ds(r * L, L)] = sa + sb

    for r in range(ROWS_PER_W):
      row_v, _ = bufs[r % 2]
      prefetch = None
      if r + 1 < ROWS_PER_W:
        nbuf, nsem = bufs[(r + 1) % 2]
        prefetch = (nbuf, nsem, rowbase + r + 1)
      make_row(row_v, r, handles[r], prefetch)

    pltpu.sync_copy(vals_v, vals_hbm.at[pl.ds(rowbase * NCAND,
                                              ROWS_PER_W * NCAND)])
    pltpu.sync_copy(ids_v, ids_hbm.at[pl.ds(rowbase * NCAND,
                                            ROWS_PER_W * NCAND)])
    pltpu.sync_copy(s_v, s_hbm.at[pl.ds(rowbase * L, ROWS_PER_W * L)])

  return sc_k(x)


def _sel_body(cv_ref, ci_ref, s_ref, bs_ref, pa_ref, it_ref,
              oh_ref, lse_ref, pred_ref, sc_ref):
  cand_vals = cv_ref[...]
  cand_ids = ci_ref[...]
  s16 = s_ref[...]
  bscores = bs_ref[...]
  pred = pa_ref[...]
  is_term = it_ref[...]

  big = jnp.int32(1 << 30)
  neg_inf = jnp.float32(-jnp.inf)

  lse = jnp.log(jnp.sum(s16, axis=2, keepdims=True))

  work = cand_vals
  tv, ti = [], []
  for _ in range(K):
    m = jnp.max(work, axis=2, keepdims=True)
    hitm = work == m
    idsel = jnp.min(jnp.where(hitm, cand_ids, big), axis=2, keepdims=True)
    tv.append(m)
    ti.append(idsel)
    work = jnp.where(hitm & (cand_ids == idsel), neg_inf, work)
  top_vals = jnp.concatenate(tv, axis=2) - lse
  top_ids = jnp.concatenate(ti, axis=2)

  slot = lax.broadcasted_iota(jnp.int32, (B, K, K), 2)
  term_scores = jnp.where(slot == 0, 0.0, neg_inf)
  top_vals = jnp.where(is_term > 0, term_scores, top_vals)
  scores = top_vals + bscores

  fidx = (lax.broadcasted_iota(jnp.int32, (B, K, K), 1) * K
          + slot)
  w = scores
  ns = jnp.zeros((B, K, 1), jnp.float32)
  sid = jnp.zeros((B, K, 1), jnp.int32)
  aid = jnp.zeros((B, K, 1), jnp.int32)
  kslot = lax.broadcasted_iota(jnp.int32, (B, K, 1), 1)
  for k in range(K):
    m2 = jnp.max(w, axis=2, keepdims=True)
    m = jnp.max(m2, axis=1, keepdims=True)
    hitm = w == m
    s2 = jnp.min(jnp.where(hitm, fidx, big), axis=2, keepdims=True)
    s = jnp.min(s2, axis=1, keepdims=True)
    a2 = jnp.sum(jnp.where(fidx == s, top_ids, 0), axis=2, keepdims=True)
    a = jnp.sum(a2, axis=1, keepdims=True)
    pick = kslot == k
    ns = jnp.where(pick, m, ns)
    sid = jnp.where(pick, s // K, sid)
    aid = jnp.where(pick, a, aid)
    w = jnp.where(fidx == s, neg_inf, w)
  sc_ref[...] = ns

  onehot = (sid == slot).astype(jnp.float32)
  oh_ref[...] = onehot

  lse_g = jnp.zeros((B, K, 1), jnp.float32)
  pred_g = pred
  for j in range(K):
    pick = sid == j
    lse_g = jnp.where(pick, lse[:, j:j + 1, :], lse_g)
    pred_g = jnp.where(pick, pred[:, j:j + 1, :], pred_g)
  lse_ref[...] = lse_g
  pred_ref[:, :, pl.ds(0, 16)] = pred_g
  pred_ref[:, :, pl.ds(16, 1)] = aid


def _sel_stage(cand_vals, cand_ids, s16, bscores, predactions, is_term):
  return pl.pallas_call(
      _sel_body,
      out_shape=[
          jax.ShapeDtypeStruct((B, K, K), jnp.float32),
          jax.ShapeDtypeStruct((B, K, 1), jnp.float32),
          jax.ShapeDtypeStruct((B, K, 17), jnp.int32),
          jax.ShapeDtypeStruct((B, K, 1), jnp.float32),
      ],
  )(cand_vals, cand_ids, s16, bscores, predactions, is_term)


def _gather_body(x_ref, oh_ref, lse_ref, out_ref):
  onehot = oh_ref[0]
  x = x_ref[0]
  lse_g = lse_ref[...]
  xg = lax.dot_general(onehot, x, (((1,), (0,)), ((), ())),
                       preferred_element_type=jnp.float32)
  out_ref[0] = xg - lse_g.reshape(K, 1)


def _gather_stage(x, onehot, lse_g):
  return pl.pallas_call(
      _gather_body,
      grid=(B,),
      in_specs=[
          pl.BlockSpec((1, K, V), lambda b: (b, 0, 0)),
          pl.BlockSpec((1, K, K), lambda b: (b, 0, 0)),
          pl.BlockSpec((1, K, 1), lambda b: (b, 0, 0)),
      ],
      out_specs=pl.BlockSpec((1, K, V), lambda b: (b, 0, 0)),
      out_shape=jax.ShapeDtypeStruct((B, K, V), jnp.float32),
  )(x, onehot, lse_g)


def kernel(actionprobs, bscores, predactions, is_term):
  x2 = actionprobs.reshape(NROWS, V)
  cand_vals, cand_ids, s16 = _sc_stage(x2)
  onehot, lse_g, new_pred, new_scores = _sel_stage(
      cand_vals.reshape(B, K, NCAND), cand_ids.reshape(B, K, NCAND),
      s16.reshape(B, K, L), bscores.reshape(B, K, 1), predactions,
      is_term.astype(jnp.int32).reshape(B, K, 1))
  out_x = _gather_stage(actionprobs, onehot, lse_g)
  return (out_x, new_pred, new_scores.reshape(B, K))


# speedup vs baseline: 41.3680x; 1.3907x over previous
import functools

import jax
import jax.numpy as jnp
from jax import lax
from jax.experimental import pallas as pl
from jax.experimental.pallas import tpu as pltpu
from jax.experimental.pallas import tpu_sc as plsc

B, K, V = 32, 8, 32000
L = 16
NROWS = B * K
NW = 32
ROWS_PER_W = NROWS // NW
NV = V // L
GROUP = 8
NG = NV // GROUP
SUPER = 5
NSG = NG // SUPER
NCAND = K * L


def _insert(Rs, Is, v, idx):
  nR, nI = [], []
  for j in range(K):
    c = v > Rs[j]
    nR.append(jnp.where(c, v, Rs[j]))
    nI.append(jnp.where(c, idx, Is[j]))
    v, idx = jnp.where(c, Rs[j], v), jnp.where(c, Is[j], idx)
  return tuple(nR), tuple(nI)


def _sc_stage(x):
  mesh = plsc.VectorSubcoreMesh(core_axis_name="c", subcore_axis_name="s")

  @functools.partial(
      pl.kernel,
      mesh=mesh,
      compiler_params=pltpu.CompilerParams(needs_layout_passes=False),
      out_type=[
          jax.ShapeDtypeStruct((NROWS * NCAND,), jnp.float32),
          jax.ShapeDtypeStruct((NROWS * NCAND,), jnp.int32),
          jax.ShapeDtypeStruct((NROWS * L,), jnp.float32),
      ],
      scratch_types=[
          pltpu.VMEM((V,), jnp.float32),
          pltpu.VMEM((V,), jnp.float32),
          pltpu.VMEM((NG * L,), jnp.float32),
          pltpu.VMEM((ROWS_PER_W * NCAND,), jnp.float32),
          pltpu.VMEM((ROWS_PER_W * NCAND,), jnp.int32),
          pltpu.VMEM((ROWS_PER_W * L,), jnp.float32),
          pltpu.SemaphoreType.DMA,
          pltpu.SemaphoreType.DMA,
      ],
  )
  def sc_k(x_hbm, vals_hbm, ids_hbm, s_hbm, row_a, row_b, gsc, vals_v,
           ids_v, s_v, sem_a, sem_b):
    wid = lax.axis_index("s") * 2 + lax.axis_index("c")
    lane = lax.iota(jnp.int32, 16)
    neg_inf = jnp.float32(-jnp.inf)
    rowbase = wid * ROWS_PER_W

    bufs = [(row_a, sem_a), (row_b, sem_b)]
    handles = [pltpu.async_copy(x_hbm.at[rowbase], row_a, sem_a)]

    def any_ge(v, th):
      cnt = plsc.all_reduce_population_count(v >= th)
      return cnt[0] > 0

    def make_row(row_v, r, handle, prefetch):
      handle.wait()
      if prefetch is not None:
        nbuf, nsem, nrow = prefetch
        handles.append(pltpu.async_copy(x_hbm.at[nrow], nbuf, nsem))

      def phase_a(g, carry):
        sa, sb, m16 = carry
        base = g * (GROUP * L)
        vs = [row_v[pl.ds(base + t * L, L)] for t in range(GROUP)]
        for t in range(0, GROUP, 2):
          sa = sa + jnp.exp(vs[t])
          sb = sb + jnp.exp(vs[t + 1])
        gmax = vs[0]
        for t in range(1, GROUP):
          gmax = jnp.maximum(gmax, vs[t])
        gsc[pl.ds(g * L, L)] = gmax
        return sa, sb, jnp.maximum(m16, gmax)

      z = jnp.zeros((L,), jnp.float32)
      ninf16 = jnp.full((L,), neg_inf, jnp.float32)
      sa, sb, m16 = lax.fori_loop(0, NG, phase_a, (z, z, ninf16))

      th = plsc.sort_key_val(m16, lane, descending=True)[0][K - 1]

      def phase_b(sgi, carry):
        gbase = sgi * SUPER
        gms = [gsc[pl.ds((gbase + u) * L, L)] for u in range(SUPER)]
        smax = gms[0]
        for u in range(1, SUPER):
          smax = jnp.maximum(smax, gms[u])

        def super_slow(args):
          def gbody(u, a):
            gm = gsc[pl.ds((gbase + u) * L, L)]

            def gslow(a2):
              def cbody(t, a3):
                ci = (gbase + u) * GROUP + t
                v = row_v[pl.ds(ci * L, L)]

                def ins(a4):
                  return _insert(a4[0], a4[1], v, lane + ci * L)

                return lax.cond(any_ge(v, th), ins, lambda a4: a4, a3)

              return lax.fori_loop(0, GROUP, cbody, a2)

            return lax.cond(any_ge(gm, th), gslow, lambda a2: a2, a)

          return lax.fori_loop(0, SUPER, gbody, args)

        return lax.cond(any_ge(smax, th), super_slow, lambda a: a, carry)

      R0 = tuple(ninf16 for _ in range(K))
      I0 = tuple(jnp.zeros((L,), jnp.int32) for _ in range(K))
      Rs, Is = lax.fori_loop(0, NSG, phase_b, (R0, I0))

      for j in range(K):
        vals_v[pl.ds(r * NCAND + j * L, L)] = Rs[j]
        ids_v[pl.ds(r * NCAND + j * L, L)] = Is[j]
      s_v[pl.ds(r * L, L)] = sa + sb

    for r in range(ROWS_PER_W):
      row_v, _ = bufs[r % 2]
      prefetch = None
      if r + 1 < ROWS_PER_W:
        nbuf, nsem = bufs[(r + 1) % 2]
        prefetch = (nbuf, nsem, rowbase + r + 1)
      make_row(row_v, r, handles[r], prefetch)

    pltpu.sync_copy(vals_v, vals_hbm.at[pl.ds(rowbase * NCAND,
                                              ROWS_PER_W * NCAND)])
    pltpu.sync_copy(ids_v, ids_hbm.at[pl.ds(rowbase * NCAND,
                                            ROWS_PER_W * NCAND)])
    pltpu.sync_copy(s_v, s_hbm.at[pl.ds(rowbase * L, ROWS_PER_W * L)])

  return sc_k(x)


def _sel_body(cv_ref, ci_ref, s_ref, bs_ref, pa_ref, it_ref,
              oh_ref, lse_ref, pred_ref, sc_ref):
  cand_vals = cv_ref[...]
  cand_ids = ci_ref[...]
  s16 = s_ref[...]
  bscores = bs_ref[...]
  pred = pa_ref[...]
  is_term = it_ref[...]

  big = jnp.int32(1 << 30)
  neg_inf = jnp.float32(-jnp.inf)

  lse = jnp.log(jnp.sum(s16, axis=2, keepdims=True))

  work = cand_vals
  tv, ti = [], []
  for _ in range(K):
    m = jnp.max(work, axis=2, keepdims=True)
    hitm = work == m
    idsel = jnp.min(jnp.where(hitm, cand_ids, big), axis=2, keepdims=True)
    tv.append(m)
    ti.append(idsel)
    work = jnp.where(hitm & (cand_ids == idsel), neg_inf, work)
  top_vals = jnp.concatenate(tv, axis=2) - lse
  top_ids = jnp.concatenate(ti, axis=2)

  slot = lax.broadcasted_iota(jnp.int32, (B, K, K), 2)
  term_scores = jnp.where(slot == 0, 0.0, neg_inf)
  top_vals = jnp.where(is_term > 0, term_scores, top_vals)
  scores = top_vals + bscores

  fidx = (lax.broadcasted_iota(jnp.int32, (B, K, K), 1) * K
          + slot)
  w = scores
  ns = jnp.zeros((B, K, 1), jnp.float32)
  sid = jnp.zeros((B, K, 1), jnp.int32)
  aid = jnp.zeros((B, K, 1), jnp.int32)
  kslot = lax.broadcasted_iota(jnp.int32, (B, K, 1), 1)
  for k in range(K):
    m2 = jnp.max(w, axis=2, keepdims=True)
    m = jnp.max(m2, axis=1, keepdims=True)
    hitm = w == m
    s2 = jnp.min(jnp.where(hitm, fidx, big), axis=2, keepdims=True)
    s = jnp.min(s2, axis=1, keepdims=True)
    a2 = jnp.sum(jnp.where(fidx == s, top_ids, 0), axis=2, keepdims=True)
    a = jnp.sum(a2, axis=1, keepdims=True)
    pick = kslot == k
    ns = jnp.where(pick, m, ns)
    sid = jnp.where(pick, s // K, sid)
    aid = jnp.where(pick, a, aid)
    w = jnp.where(fidx == s, neg_inf, w)
  sc_ref[...] = ns

  onehot = (sid == slot).astype(jnp.float32)
  oh_ref[...] = onehot

  lse_g = jnp.zeros((B, K, 1), jnp.float32)
  pred_g = pred
  for j in range(K):
    pick = sid == j
    lse_g = jnp.where(pick, lse[:, j:j + 1, :], lse_g)
    pred_g = jnp.where(pick, pred[:, j:j + 1, :], pred_g)
  lse_ref[...] = lse_g
  pred_ref[:, :, pl.ds(0, 16)] = pred_g
  pred_ref[:, :, pl.ds(16, 1)] = aid


def _sel_stage(cand_vals, cand_ids, s16, bscores, predactions, is_term):
  return pl.pallas_call(
      _sel_body,
      out_shape=[
          jax.ShapeDtypeStruct((B, K, K), jnp.float32),
          jax.ShapeDtypeStruct((B, K, 1), jnp.float32),
          jax.ShapeDtypeStruct((B, K, 17), jnp.int32),
          jax.ShapeDtypeStruct((B, K, 1), jnp.float32),
      ],
  )(cand_vals, cand_ids, s16, bscores, predactions, is_term)


def _gather_body(x_ref, oh_ref, lse_ref, out_ref):
  onehot = oh_ref[0]
  x = x_ref[0]
  lse_g = lse_ref[...]
  xg = lax.dot_general(onehot, x, (((1,), (0,)), ((), ())),
                       preferred_element_type=jnp.float32)
  out_ref[0] = xg - lse_g.reshape(K, 1)


def _gather_stage(x, onehot, lse_g):
  return pl.pallas_call(
      _gather_body,
      grid=(B,),
      in_specs=[
          pl.BlockSpec((1, K, V), lambda b: (b, 0, 0)),
          pl.BlockSpec((1, K, K), lambda b: (b, 0, 0)),
          pl.BlockSpec((1, K, 1), lambda b: (b, 0, 0)),
      ],
      out_specs=pl.BlockSpec((1, K, V), lambda b: (b, 0, 0)),
      out_shape=jax.ShapeDtypeStruct((B, K, V), jnp.float32),
  )(x, onehot, lse_g)


def kernel(actionprobs, bscores, predactions, is_term):
  x2 = actionprobs.reshape(NROWS, V)
  cand_vals, cand_ids, s16 = _sc_stage(x2)
  onehot, lse_g, new_pred, new_scores = _sel_stage(
      cand_vals.reshape(B, K, NCAND), cand_ids.reshape(B, K, NCAND),
      s16.reshape(B, K, L), bscores.reshape(B, K, 1), predactions,
      is_term.astype(jnp.int32).reshape(B, K, 1))
  out_x = _gather_stage(actionprobs, onehot, lse_g)
  return (out_x, new_pred, new_scores.reshape(B, K))


# speedup vs baseline: 41.9464x; 1.0140x over previous
import functools

import jax
import jax.numpy as jnp
from jax import lax
from jax.experimental import pallas as pl
from jax.experimental.pallas import tpu as pltpu
from jax.experimental.pallas import tpu_sc as plsc

B, K, V = 32, 8, 32000
L = 16
NROWS = B * K
NW = 32
ROWS_PER_W = NROWS // NW
NV = V // L
GROUP = 8
NG = NV // GROUP
SUPER = 5
NSG = NG // SUPER
NCAND = K * L


def _insert(Rs, Is, v, idx):
  nR, nI = [], []
  for j in range(K):
    c = v > Rs[j]
    nR.append(jnp.where(c, v, Rs[j]))
    nI.append(jnp.where(c, idx, Is[j]))
    v, idx = jnp.where(c, Rs[j], v), jnp.where(c, Is[j], idx)
  return tuple(nR), tuple(nI)


def _sc_stage(x):
  mesh = plsc.VectorSubcoreMesh(core_axis_name="c", subcore_axis_name="s")

  @functools.partial(
      pl.kernel,
      mesh=mesh,
      compiler_params=pltpu.CompilerParams(needs_layout_passes=False),
      out_type=[
          jax.ShapeDtypeStruct((NROWS * NCAND,), jnp.float32),
          jax.ShapeDtypeStruct((NROWS * NCAND,), jnp.int32),
          jax.ShapeDtypeStruct((NROWS * L,), jnp.float32),
      ],
      scratch_types=[
          pltpu.VMEM((V,), jnp.float32),
          pltpu.VMEM((V,), jnp.float32),
          pltpu.VMEM((NG * L,), jnp.float32),
          pltpu.VMEM((ROWS_PER_W * NCAND,), jnp.float32),
          pltpu.VMEM((ROWS_PER_W * NCAND,), jnp.int32),
          pltpu.VMEM((ROWS_PER_W * L,), jnp.float32),
          pltpu.SemaphoreType.DMA,
          pltpu.SemaphoreType.DMA,
      ],
  )
  def sc_k(x_hbm, vals_hbm, ids_hbm, s_hbm, row_a, row_b, gsc, vals_v,
           ids_v, s_v, sem_a, sem_b):
    wid = lax.axis_index("s") * 2 + lax.axis_index("c")
    lane = lax.iota(jnp.int32, 16)
    neg_inf = jnp.float32(-jnp.inf)
    rowbase = wid * ROWS_PER_W

    bufs = [(row_a, sem_a), (row_b, sem_b)]
    handles = [pltpu.async_copy(x_hbm.at[rowbase], row_a, sem_a)]

    def any_ge(v, th):
      cnt = plsc.all_reduce_population_count(v >= th)
      return cnt[0] > 0

    def make_row(row_v, r, handle, prefetch):
      handle.wait()
      if prefetch is not None:
        nbuf, nsem, nrow = prefetch
        handles.append(pltpu.async_copy(x_hbm.at[nrow], nbuf, nsem))

      def phase_a(h, carry):
        sa, sb, m16 = carry
        for u in range(2):
          g = h * 2 + u
          base = g * (GROUP * L)
          vs = [row_v[pl.ds(base + t * L, L)] for t in range(GROUP)]
          for t in range(0, GROUP, 2):
            sa = sa + jnp.exp(vs[t])
            sb = sb + jnp.exp(vs[t + 1])
          gmax = vs[0]
          for t in range(1, GROUP):
            gmax = jnp.maximum(gmax, vs[t])
          gsc[pl.ds(g * L, L)] = gmax
          m16 = jnp.maximum(m16, gmax)
        return sa, sb, m16

      z = jnp.zeros((L,), jnp.float32)
      ninf16 = jnp.full((L,), neg_inf, jnp.float32)
      sa, sb, m16 = lax.fori_loop(0, NG // 2, phase_a, (z, z, ninf16))

      th = plsc.sort_key_val(m16, lane, descending=True)[0][K - 1]

      def phase_b(sgi, carry):
        gbase = sgi * SUPER
        gms = [gsc[pl.ds((gbase + u) * L, L)] for u in range(SUPER)]
        smax = gms[0]
        for u in range(1, SUPER):
          smax = jnp.maximum(smax, gms[u])

        def super_slow(args):
          def gbody(u, a):
            gm = gsc[pl.ds((gbase + u) * L, L)]

            def gslow(a2):
              def cbody(t, a3):
                ci = (gbase + u) * GROUP + t
                v = row_v[pl.ds(ci * L, L)]

                def ins(a4):
                  return _insert(a4[0], a4[1], v, lane + ci * L)

                return lax.cond(any_ge(v, th), ins, lambda a4: a4, a3)

              return lax.fori_loop(0, GROUP, cbody, a2)

            return lax.cond(any_ge(gm, th), gslow, lambda a2: a2, a)

          return lax.fori_loop(0, SUPER, gbody, args)

        return lax.cond(any_ge(smax, th), super_slow, lambda a: a, carry)

      R0 = tuple(ninf16 for _ in range(K))
      I0 = tuple(jnp.zeros((L,), jnp.int32) for _ in range(K))
      Rs, Is = lax.fori_loop(0, NSG, phase_b, (R0, I0))

      for j in range(K):
        vals_v[pl.ds(r * NCAND + j * L, L)] = Rs[j]
        ids_v[pl.ds(r * NCAND + j * L, L)] = Is[j]
      s_v[pl.ds(r * L, L)] = sa + sb

    for r in range(ROWS_PER_W):
      row_v, _ = bufs[r % 2]
      prefetch = None
      if r + 1 < ROWS_PER_W:
        nbuf, nsem = bufs[(r + 1) % 2]
        prefetch = (nbuf, nsem, rowbase + r + 1)
      make_row(row_v, r, handles[r], prefetch)

    pltpu.sync_copy(vals_v, vals_hbm.at[pl.ds(rowbase * NCAND,
                                              ROWS_PER_W * NCAND)])
    pltpu.sync_copy(ids_v, ids_hbm.at[pl.ds(rowbase * NCAND,
                                            ROWS_PER_W * NCAND)])
    pltpu.sync_copy(s_v, s_hbm.at[pl.ds(rowbase * L, ROWS_PER_W * L)])

  return sc_k(x)


def _sel_compute(cv_ref, ci_ref, s_ref, bs_ref, pa_ref, it_ref,
                 oh_ref, lse_ref, pred_ref, sc_ref):
  cand_vals = cv_ref[...]
  cand_ids = ci_ref[...]
  s16 = s_ref[...]
  bscores = bs_ref[...]
  pred = pa_ref[...]
  is_term = it_ref[...]

  big = jnp.int32(1 << 30)
  neg_inf = jnp.float32(-jnp.inf)

  lse = jnp.log(jnp.sum(s16, axis=2, keepdims=True))

  work = cand_vals
  tv, ti = [], []
  for _ in range(K):
    m = jnp.max(work, axis=2, keepdims=True)
    hitm = work == m
    idsel = jnp.min(jnp.where(hitm, cand_ids, big), axis=2, keepdims=True)
    tv.append(m)
    ti.append(idsel)
    work = jnp.where(hitm & (cand_ids == idsel), neg_inf, work)
  top_vals = jnp.concatenate(tv, axis=2) - lse
  top_ids = jnp.concatenate(ti, axis=2)

  slot = lax.broadcasted_iota(jnp.int32, (B, K, K), 2)
  term_scores = jnp.where(slot == 0, 0.0, neg_inf)
  top_vals = jnp.where(is_term > 0, term_scores, top_vals)
  scores = top_vals + bscores

  fidx = (lax.broadcasted_iota(jnp.int32, (B, K, K), 1) * K
          + slot)
  w = scores
  ns = jnp.zeros((B, K, 1), jnp.float32)
  sid = jnp.zeros((B, K, 1), jnp.int32)
  aid = jnp.zeros((B, K, 1), jnp.int32)
  kslot = lax.broadcasted_iota(jnp.int32, (B, K, 1), 1)
  for k in range(K):
    m2 = jnp.max(w, axis=2, keepdims=True)
    m = jnp.max(m2, axis=1, keepdims=True)
    hitm = w == m
    s2 = jnp.min(jnp.where(hitm, fidx, big), axis=2, keepdims=True)
    s = jnp.min(s2, axis=1, keepdims=True)
    a2 = jnp.sum(jnp.where(fidx == s, top_ids, 0), axis=2, keepdims=True)
    a = jnp.sum(a2, axis=1, keepdims=True)
    pick = kslot == k
    ns = jnp.where(pick, m, ns)
    sid = jnp.where(pick, s // K, sid)
    aid = jnp.where(pick, a, aid)
    w = jnp.where(fidx == s, neg_inf, w)
  sc_ref[...] = ns

  onehot = (sid == slot).astype(jnp.float32)
  oh_ref[...] = onehot

  lse_g = jnp.zeros((B, K, 1), jnp.float32)
  pred_g = pred
  for j in range(K):
    pick = sid == j
    lse_g = jnp.where(pick, lse[:, j:j + 1, :], lse_g)
    pred_g = jnp.where(pick, pred[:, j:j + 1, :], pred_g)
  lse_ref[...] = lse_g
  pred_ref[:, :, pl.ds(0, 16)] = pred_g
  pred_ref[:, :, pl.ds(16, 1)] = aid


def _tc_body(cv_ref, ci_ref, s_ref, bs_ref, pa_ref, it_ref, x_ref,
             out_ref, pred_ref, sc_ref, oh_sc, lse_sc):
  b = pl.program_id(0)

  @pl.when(b == 0)
  def _():
    _sel_compute(cv_ref, ci_ref, s_ref, bs_ref, pa_ref, it_ref,
                 oh_sc, lse_sc, pred_ref, sc_ref)

  onehot = oh_sc[b]
  x = x_ref[0]
  lse_g = lse_sc[b]
  xg = lax.dot_general(onehot, x, (((1,), (0,)), ((), ())),
                       preferred_element_type=jnp.float32)
  out_ref[0] = xg - lse_g


def _tc_stage(cand_vals, cand_ids, s16, bscores, predactions, is_term, x):
  fix = lambda b: (0, 0, 0)
  bmap = lambda b: (b, 0, 0)
  return pl.pallas_call(
      _tc_body,
      grid=(B,),
      in_specs=[
          pl.BlockSpec((B, K, NCAND), fix),
          pl.BlockSpec((B, K, NCAND), fix),
          pl.BlockSpec((B, K, L), fix),
          pl.BlockSpec((B, K, 1), fix),
          pl.BlockSpec((B, K, 16), fix),
          pl.BlockSpec((B, K, 1), fix),
          pl.BlockSpec((1, K, V), bmap),
      ],
      out_specs=[
          pl.BlockSpec((1, K, V), bmap),
          pl.BlockSpec((B, K, 17), fix),
          pl.BlockSpec((B, K, 1), fix),
      ],
      out_shape=[
          jax.ShapeDtypeStruct((B, K, V), jnp.float32),
          jax.ShapeDtypeStruct((B, K, 17), jnp.int32),
          jax.ShapeDtypeStruct((B, K, 1), jnp.float32),
      ],
      scratch_shapes=[
          pltpu.VMEM((B, K, K), jnp.float32),
          pltpu.VMEM((B, K, 1), jnp.float32),
      ],
  )(cand_vals, cand_ids, s16, bscores, predactions, is_term, x)


def kernel(actionprobs, bscores, predactions, is_term):
  x2 = actionprobs.reshape(NROWS, V)
  cand_vals, cand_ids, s16 = _sc_stage(x2)
  out_x, new_pred, new_scores = _tc_stage(
      cand_vals.reshape(B, K, NCAND), cand_ids.reshape(B, K, NCAND),
      s16.reshape(B, K, L), bscores.reshape(B, K, 1), predactions,
      is_term.astype(jnp.int32).reshape(B, K, 1), actionprobs)
  return (out_x, new_pred, new_scores.reshape(B, K))
